# baseline (speedup 1.0000x reference)
import functools
import math

import jax
import jax.numpy as jnp
from jax import lax
from jax.experimental import pallas as pl
from jax.experimental.pallas import tpu as pltpu
from jax.experimental.pallas import tpu_sc as plsc

_STATUS = 128
_QL = 50
_EF = 16
_EMD = _QL * _EF
_H = 4
_DH = 32
_GF = 1600
_HID = 64
_OUT = 64

_DP = lax.Precision.DEFAULT

_NW = 32
_CH = 128


def _gumbel_noise(bsz):
    u = jax.random.uniform(jax.random.key(42), (bsz, _QL), dtype=jnp.float32)
    return -jnp.log(-jnp.log(u + 1e-20) + 1e-20)


def _t_body(vs_ref, wqt_ref, wk_ref, t_ref):
    qf_t = jnp.dot(wqt_ref[...], vs_ref[...], precision=_DP)
    t_ref[...] = jnp.dot(wk_ref[...], qf_t, precision=_DP) / jnp.float32(
        _H * math.sqrt(_DH))


def _hpart_body(vs_ref, img_ref, w1t_ref, b1_ref, h_ref):
    w1t = w1t_ref[...]
    h_ref[...] = (jnp.dot(w1t[:, :_STATUS], vs_ref[...], precision=_DP)
                  + jnp.dot(w1t[:, _STATUS + _EF:], img_ref[...], precision=_DP)
                  + b1_ref[...])


def _out_body(h_ref, sel_ref, w1t_ref, w2t_ref, b2_ref, out_ref):
    h = jnp.maximum(
        h_ref[...]
        + jnp.dot(w1t_ref[...][:, _STATUS:_STATUS + _EF], sel_ref[...],
                  precision=_DP), 0.0)
    out_ref[...] = jnp.maximum(
        jnp.dot(w2t_ref[...], h, precision=_DP) + b2_ref[...], 0.0)


def _sc_routing_body(vst_hbm, tt_hbm, g_hbm, out_hbm, em_v, t_v, g_v, wm_v,
                     sel_v):
    nc = 2
    wid = lax.axis_index("s") * nc + lax.axis_index("c")
    bsz = vst_hbm.shape[1]
    cpw = bsz // _NW
    base = wid * cpw

    def chunk_body(ci, carry):
        c0 = base + ci * _CH
        pltpu.sync_copy(vst_hbm.at[pl.ds(_STATUS, _EMD), pl.ds(c0, _CH)], em_v)
        pltpu.sync_copy(tt_hbm.at[:, pl.ds(c0, _CH)], t_v)
        pltpu.sync_copy(g_hbm.at[:, pl.ds(c0, _CH)], g_v)
        for lg in range(_CH // 16):
            lanes = pl.ds(lg * 16, 16)
            tv = [t_v[f, lanes] for f in range(_EF)]

            def wm_body(q, c):
                acc = jnp.zeros((16,), jnp.float32)
                ma = jnp.zeros((16,), jnp.float32)
                for f in range(_EF):
                    v = em_v[q * _EF + f, lanes]
                    acc = acc + v * tv[f]
                    ma = jnp.maximum(ma, jnp.abs(v))
                wm = jnp.where(ma == 0.0, jnp.float32(-1e8), acc)
                wm_v[q, lanes] = (wm + jnp.float32(1e-8)
                                  + g_v[q, lanes]) / jnp.float32(0.1)
                return c

            lax.fori_loop(0, _QL, wm_body, 0)

            def max_body(q, m):
                return jnp.maximum(m, wm_v[q, lanes])

            m = lax.fori_loop(0, _QL, max_body,
                              jnp.full((16,), -jnp.inf, jnp.float32))

            def exp_body(q, s):
                e = jnp.exp(wm_v[q, lanes] - m)
                wm_v[q, lanes] = e
                return s + e

            s = lax.fori_loop(0, _QL, exp_body, jnp.zeros((16,), jnp.float32))
            inv = 1.0 / s

            def sel_body(q, accs):
                w = wm_v[q, lanes] * inv
                return tuple(accs[f] + w * em_v[q * _EF + f, lanes]
                             for f in range(_EF))

            accs = lax.fori_loop(
                0, _QL, sel_body,
                tuple(jnp.zeros((16,), jnp.float32) for _ in range(_EF)))
            for f in range(_EF):
                sel_v[f, lanes] = accs[f]
        pltpu.sync_copy(sel_v, out_hbm.at[:, pl.ds(c0, _CH)])
        return carry

    lax.fori_loop(0, cpw // _CH, chunk_body, 0)


@jax.jit
def kernel(vector_state, image_state, Wq, Wk, Wv, W1, b1, W2, b2):
    del Wv
    bsz = vector_state.shape[0]
    vst = vector_state.T
    imgt = image_state.transpose(1, 2, 3, 0).reshape(_GF, bsz)
    gt = _gumbel_noise(bsz).T
    wqt = Wq.T
    w1t = W1.T
    w2t = W2.T
    b1c = b1.reshape(_HID, 1)
    b2c = b2.reshape(_OUT, 1)

    btl = 512
    grid = (bsz // btl,)
    col = lambda i: (0, i)
    rep = lambda i: (0, 0)

    t_t = pl.pallas_call(
        _t_body,
        grid=grid,
        in_specs=[
            pl.BlockSpec((_STATUS, btl), col),
            pl.BlockSpec(wqt.shape, rep),
            pl.BlockSpec(Wk.shape, rep),
        ],
        out_specs=pl.BlockSpec((_EF, btl), col),
        out_shape=jax.ShapeDtypeStruct((_EF, bsz), jnp.float32),
    )(vst, wqt, Wk)

    sc_routing = functools.partial(
        pl.kernel,
        mesh=plsc.VectorSubcoreMesh(core_axis_name="c", subcore_axis_name="s"),
        out_type=jax.ShapeDtypeStruct((_EF, bsz), jnp.float32),
        scratch_types=[
            pltpu.VMEM((_EMD, _CH), jnp.float32),
            pltpu.VMEM((_EF, _CH), jnp.float32),
            pltpu.VMEM((_QL, _CH), jnp.float32),
            pltpu.VMEM((_QL, _CH), jnp.float32),
            pltpu.VMEM((_EF, _CH), jnp.float32),
        ],
    )(_sc_routing_body)
    selected_t = sc_routing(vst, t_t, gt)

    hpart = pl.pallas_call(
        _hpart_body,
        grid=grid,
        in_specs=[
            pl.BlockSpec((_STATUS, btl), col),
            pl.BlockSpec((_GF, btl), col),
            pl.BlockSpec(w1t.shape, rep),
            pl.BlockSpec(b1c.shape, rep),
        ],
        out_specs=pl.BlockSpec((_HID, btl), col),
        out_shape=jax.ShapeDtypeStruct((_HID, bsz), jnp.float32),
    )(vst, imgt, w1t, b1c)

    out_t = pl.pallas_call(
        _out_body,
        grid=grid,
        in_specs=[
            pl.BlockSpec((_HID, btl), col),
            pl.BlockSpec((_EF, btl), col),
            pl.BlockSpec(w1t.shape, rep),
            pl.BlockSpec(w2t.shape, rep),
            pl.BlockSpec(b2c.shape, rep),
        ],
        out_specs=pl.BlockSpec((_OUT, btl), col),
        out_shape=jax.ShapeDtypeStruct((_OUT, bsz), jnp.float32),
    )(hpart, selected_t, w1t, w2t, b2c)
    return out_t.T


# speedup vs baseline: 1.5353x; 1.5353x over previous
import functools
import math

import jax
import jax.numpy as jnp
from jax import lax
from jax.experimental import pallas as pl
from jax.experimental.pallas import tpu as pltpu
from jax.experimental.pallas import tpu_sc as plsc

_STATUS = 128
_QL = 50
_EF = 16
_EMD = _QL * _EF
_H = 4
_DH = 32
_GF = 1600
_HID = 64
_OUT = 64

_DP = lax.Precision.DEFAULT

_NW = 32
_CH = 128
_SC_FRAC = 4


def _gumbel_noise(bsz):
    u = jax.random.uniform(jax.random.key(42), (bsz, _QL), dtype=jnp.float32)
    return -jnp.log(-jnp.log(u + 1e-20) + 1e-20)


def _routing_t(vst, wqt, wk):
    qf_t = jnp.dot(wqt, vst[:_STATUS], precision=_DP)
    return jnp.dot(wk, qf_t, precision=_DP) / jnp.float32(_H * math.sqrt(_DH))


def _select(em_t, t_t, g):
    col_f = lax.broadcasted_iota(jnp.int32, (_EMD, _EF), 0)
    row_f = lax.broadcasted_iota(jnp.int32, (_EMD, _EF), 1)
    rep = (lax.rem(col_f, _EF) == row_f).astype(jnp.float32)
    q_c = lax.broadcasted_iota(jnp.int32, (_QL, _EMD), 1)
    q_r = lax.broadcasted_iota(jnp.int32, (_QL, _EMD), 0)
    seg = (q_c // _EF == q_r).astype(jnp.float32)

    trep_t = jnp.dot(rep, t_t, precision=_DP)
    wm_t = jnp.dot(seg, em_t * trep_t, precision=_DP)
    nz = (em_t != 0.0).astype(jnp.float32)
    cnt = jnp.dot(seg, nz, precision=_DP)
    wm_t = jnp.where(cnt == 0.0, jnp.float32(-1e8), wm_t)

    logits = (wm_t + jnp.float32(1e-8) + g) / jnp.float32(0.1)
    m = jnp.max(logits, axis=0, keepdims=True)
    e = jnp.exp(logits - m)
    sel = e / jnp.sum(e, axis=0, keepdims=True)

    selrep_t = jnp.dot(seg.T, sel, precision=_DP)
    return jnp.dot(rep.T, em_t * selrep_t, precision=_DP)


def _fused_body(vs_ref, img_ref, g_ref, wqt_ref, wk_ref, w1t_ref, b1_ref,
                w2t_ref, b2_ref, out_ref):
    vst = vs_ref[...]
    t_t = _routing_t(vst, wqt_ref[...], wk_ref[...])
    selected_t = _select(vst[_STATUS:], t_t, g_ref[...])
    w1t = w1t_ref[...]
    h = (jnp.dot(w1t[:, :_STATUS], vst[:_STATUS], precision=_DP)
         + jnp.dot(w1t[:, _STATUS:_STATUS + _EF], selected_t, precision=_DP)
         + jnp.dot(w1t[:, _STATUS + _EF:], img_ref[...], precision=_DP)
         + b1_ref[...])
    h = jnp.maximum(h, 0.0)
    out_ref[...] = jnp.maximum(
        jnp.dot(w2t_ref[...], h, precision=_DP) + b2_ref[...], 0.0)


def _t_body(vs_ref, wqt_ref, wk_ref, t_ref):
    t_ref[...] = _routing_t(vs_ref[...], wqt_ref[...], wk_ref[...])


def _hpart_body(vs_ref, img_ref, w1t_ref, b1_ref, h_ref):
    w1t = w1t_ref[...]
    h_ref[...] = (jnp.dot(w1t[:, :_STATUS], vs_ref[...], precision=_DP)
                  + jnp.dot(w1t[:, _STATUS + _EF:], img_ref[...], precision=_DP)
                  + b1_ref[...])


def _out_body(h_ref, sel_ref, w1t_ref, w2t_ref, b2_ref, out_ref):
    h = jnp.maximum(
        h_ref[...]
        + jnp.dot(w1t_ref[...][:, _STATUS:_STATUS + _EF], sel_ref[...],
                  precision=_DP), 0.0)
    out_ref[...] = jnp.maximum(
        jnp.dot(w2t_ref[...], h, precision=_DP) + b2_ref[...], 0.0)


def _sc_routing_body(vst_hbm, tt_hbm, g_hbm, out_hbm, em_v, t_v, g_v, wm_v,
                     sel_v):
    nc = 2
    wid = lax.axis_index("s") * nc + lax.axis_index("c")
    n_sc = out_hbm.shape[1]
    sc0 = vst_hbm.shape[1] - n_sc
    cpw = n_sc // _NW
    base = wid * cpw

    def chunk_body(ci, carry):
        c0 = base + ci * _CH
        pltpu.sync_copy(
            vst_hbm.at[pl.ds(_STATUS, _EMD), pl.ds(sc0 + c0, _CH)], em_v)
        pltpu.sync_copy(tt_hbm.at[:, pl.ds(c0, _CH)], t_v)
        pltpu.sync_copy(g_hbm.at[:, pl.ds(sc0 + c0, _CH)], g_v)
        for lg in range(_CH // 16):
            lanes = pl.ds(lg * 16, 16)
            tv = [t_v[f, lanes] for f in range(_EF)]

            def wm_body(q, c):
                acc = jnp.zeros((16,), jnp.float32)
                ma = jnp.zeros((16,), jnp.float32)
                for f in range(_EF):
                    v = em_v[q * _EF + f, lanes]
                    acc = acc + v * tv[f]
                    ma = jnp.maximum(ma, jnp.abs(v))
                wm = jnp.where(ma == 0.0, jnp.float32(-1e8), acc)
                wm_v[q, lanes] = (wm + jnp.float32(1e-8)
                                  + g_v[q, lanes]) / jnp.float32(0.1)
                return c

            lax.fori_loop(0, _QL, wm_body, 0)

            def max_body(q, m):
                return jnp.maximum(m, wm_v[q, lanes])

            m = lax.fori_loop(0, _QL, max_body,
                              jnp.full((16,), -jnp.inf, jnp.float32))

            def exp_body(q, s):
                e = jnp.exp(wm_v[q, lanes] - m)
                wm_v[q, lanes] = e
                return s + e

            s = lax.fori_loop(0, _QL, exp_body, jnp.zeros((16,), jnp.float32))
            inv = 1.0 / s

            def sel_body(q, accs):
                w = wm_v[q, lanes] * inv
                return tuple(accs[f] + w * em_v[q * _EF + f, lanes]
                             for f in range(_EF))

            accs = lax.fori_loop(
                0, _QL, sel_body,
                tuple(jnp.zeros((16,), jnp.float32) for _ in range(_EF)))
            for f in range(_EF):
                sel_v[f, lanes] = accs[f]
        pltpu.sync_copy(sel_v, out_hbm.at[:, pl.ds(c0, _CH)])
        return carry

    lax.fori_loop(0, cpw // _CH, chunk_body, 0)


@jax.jit
def kernel(vector_state, image_state, Wq, Wk, Wv, W1, b1, W2, b2):
    del Wv
    bsz = vector_state.shape[0]
    vst = vector_state.T
    imgt = image_state.transpose(1, 2, 3, 0).reshape(_GF, bsz)
    gt = _gumbel_noise(bsz).T
    wqt = Wq.T
    w1t = W1.T
    w2t = W2.T
    b1c = b1.reshape(_HID, 1)
    b2c = b2.reshape(_OUT, 1)

    btl = 512
    n_sc = bsz // _SC_FRAC
    n_tc = bsz - n_sc
    tc_blocks = n_tc // btl
    sc_blocks = n_sc // btl
    col = lambda i: (0, i)
    sc_col = lambda i: (0, tc_blocks + i)
    rep = lambda i: (0, 0)

    t_sc = pl.pallas_call(
        _t_body,
        grid=(sc_blocks,),
        in_specs=[
            pl.BlockSpec((_STATUS, btl), sc_col),
            pl.BlockSpec(wqt.shape, rep),
            pl.BlockSpec(Wk.shape, rep),
        ],
        out_specs=pl.BlockSpec((_EF, btl), col),
        out_shape=jax.ShapeDtypeStruct((_EF, n_sc), jnp.float32),
    )(vst, wqt, Wk)

    sc_routing = functools.partial(
        pl.kernel,
        mesh=plsc.VectorSubcoreMesh(core_axis_name="c", subcore_axis_name="s"),
        out_type=jax.ShapeDtypeStruct((_EF, n_sc), jnp.float32),
        scratch_types=[
            pltpu.VMEM((_EMD, _CH), jnp.float32),
            pltpu.VMEM((_EF, _CH), jnp.float32),
            pltpu.VMEM((_QL, _CH), jnp.float32),
            pltpu.VMEM((_QL, _CH), jnp.float32),
            pltpu.VMEM((_EF, _CH), jnp.float32),
        ],
    )(_sc_routing_body)
    selected_sc = sc_routing(vst, t_sc, gt)

    out_tc = pl.pallas_call(
        _fused_body,
        grid=(tc_blocks,),
        in_specs=[
            pl.BlockSpec((_STATUS + _EMD, btl), col),
            pl.BlockSpec((_GF, btl), col),
            pl.BlockSpec((_QL, btl), col),
            pl.BlockSpec(wqt.shape, rep),
            pl.BlockSpec(Wk.shape, rep),
            pl.BlockSpec(w1t.shape, rep),
            pl.BlockSpec(b1c.shape, rep),
            pl.BlockSpec(w2t.shape, rep),
            pl.BlockSpec(b2c.shape, rep),
        ],
        out_specs=pl.BlockSpec((_OUT, btl), col),
        out_shape=jax.ShapeDtypeStruct((_OUT, n_tc), jnp.float32),
    )(vst, imgt, gt, wqt, Wk, w1t, b1c, w2t, b2c)

    hpart_sc = pl.pallas_call(
        _hpart_body,
        grid=(sc_blocks,),
        in_specs=[
            pl.BlockSpec((_STATUS, btl), sc_col),
            pl.BlockSpec((_GF, btl), sc_col),
            pl.BlockSpec(w1t.shape, rep),
            pl.BlockSpec(b1c.shape, rep),
        ],
        out_specs=pl.BlockSpec((_HID, btl), col),
        out_shape=jax.ShapeDtypeStruct((_HID, n_sc), jnp.float32),
    )(vst, imgt, w1t, b1c)

    out_sc = pl.pallas_call(
        _out_body,
        grid=(sc_blocks,),
        in_specs=[
            pl.BlockSpec((_HID, btl), col),
            pl.BlockSpec((_EF, btl), col),
            pl.BlockSpec(w1t.shape, rep),
            pl.BlockSpec(w2t.shape, rep),
            pl.BlockSpec(b2c.shape, rep),
        ],
        out_specs=pl.BlockSpec((_OUT, btl), col),
        out_shape=jax.ShapeDtypeStruct((_OUT, n_sc), jnp.float32),
    )(hpart_sc, selected_sc, w1t, w2t, b2c)

    return jnp.concatenate([out_tc, out_sc], axis=1).T


# speedup vs baseline: 1.6093x; 1.0482x over previous
import functools
import math

import jax
import jax.numpy as jnp
from jax import lax
from jax.experimental import pallas as pl
from jax.experimental.pallas import tpu as pltpu
from jax.experimental.pallas import tpu_sc as plsc

_STATUS = 128
_QL = 50
_EF = 16
_EMD = _QL * _EF
_H = 4
_DH = 32
_GF = 1600
_HID = 64
_OUT = 64

_DP = lax.Precision.DEFAULT

_NW = 32
_CH = 128
_SC_FRAC = 4


def _gumbel_noise(bsz):
    u = jax.random.uniform(jax.random.key(42), (bsz, _QL), dtype=jnp.float32)
    return -jnp.log(-jnp.log(u + 1e-20) + 1e-20)


def _routing_t(vst, wqt, wk):
    qf_t = jnp.dot(wqt, vst[:_STATUS], precision=_DP)
    return jnp.dot(wk, qf_t, precision=_DP) / jnp.float32(_H * math.sqrt(_DH))


def _select(em_t, t_t, g):
    col_f = lax.broadcasted_iota(jnp.int32, (_EMD, _EF), 0)
    row_f = lax.broadcasted_iota(jnp.int32, (_EMD, _EF), 1)
    rep = (lax.rem(col_f, _EF) == row_f).astype(jnp.float32)
    q_c = lax.broadcasted_iota(jnp.int32, (_QL, _EMD), 1)
    q_r = lax.broadcasted_iota(jnp.int32, (_QL, _EMD), 0)
    seg = (q_c // _EF == q_r).astype(jnp.float32)

    trep_t = jnp.dot(rep, t_t, precision=_DP)
    wm_t = jnp.dot(seg, em_t * trep_t, precision=_DP)
    nz = (em_t != 0.0).astype(jnp.float32)
    cnt = jnp.dot(seg, nz, precision=_DP)
    wm_t = jnp.where(cnt == 0.0, jnp.float32(-1e8), wm_t)

    logits = (wm_t + jnp.float32(1e-8) + g) / jnp.float32(0.1)
    m = jnp.max(logits, axis=0, keepdims=True)
    e = jnp.exp(logits - m)
    sel = e / jnp.sum(e, axis=0, keepdims=True)

    selrep_t = jnp.dot(seg.T, sel, precision=_DP)
    return jnp.dot(rep.T, em_t * selrep_t, precision=_DP)


def _fused_body(vs_ref, img_ref, g_ref, wqt_ref, wk_ref, w1t_ref, b1_ref,
                w2t_ref, b2_ref, out_ref):
    vst = vs_ref[...]
    t_t = _routing_t(vst, wqt_ref[...], wk_ref[...])
    selected_t = _select(vst[_STATUS:], t_t, g_ref[...])
    w1t = w1t_ref[...]
    h = (jnp.dot(w1t[:, :_STATUS], vst[:_STATUS], precision=_DP)
         + jnp.dot(w1t[:, _STATUS:_STATUS + _EF], selected_t, precision=_DP)
         + jnp.dot(w1t[:, _STATUS + _EF:], img_ref[...], precision=_DP)
         + b1_ref[...])
    h = jnp.maximum(h, 0.0)
    out_ref[...] = jnp.maximum(
        jnp.dot(w2t_ref[...], h, precision=_DP) + b2_ref[...], 0.0)


def _t_body(vs_ref, wqt_ref, wk_ref, t_ref):
    t_ref[...] = _routing_t(vs_ref[...], wqt_ref[...], wk_ref[...])


def _mlp_body(vs_ref, img_ref, sel_ref, w1t_ref, b1_ref, w2t_ref, b2_ref,
              out_ref):
    w1t = w1t_ref[...]
    h = jnp.maximum(
        jnp.dot(w1t[:, :_STATUS], vs_ref[...], precision=_DP)
        + jnp.dot(w1t[:, _STATUS:_STATUS + _EF], sel_ref[...], precision=_DP)
        + jnp.dot(w1t[:, _STATUS + _EF:], img_ref[...], precision=_DP)
        + b1_ref[...], 0.0)
    out_ref[...] = jnp.maximum(
        jnp.dot(w2t_ref[...], h, precision=_DP) + b2_ref[...], 0.0)


def _sc_routing_body(vst_hbm, tt_hbm, g_hbm, out_hbm, em_v, t_v, g_v, wm_v,
                     sel_v):
    nc = 2
    wid = lax.axis_index("s") * nc + lax.axis_index("c")
    n_sc = out_hbm.shape[1]
    sc0 = vst_hbm.shape[1] - n_sc
    cpw = n_sc // _NW
    base = wid * cpw

    def chunk_body(ci, carry):
        c0 = base + ci * _CH
        pltpu.sync_copy(
            vst_hbm.at[pl.ds(_STATUS, _EMD), pl.ds(sc0 + c0, _CH)], em_v)
        pltpu.sync_copy(tt_hbm.at[:, pl.ds(c0, _CH)], t_v)
        pltpu.sync_copy(g_hbm.at[:, pl.ds(sc0 + c0, _CH)], g_v)
        for lg in range(_CH // 16):
            lanes = pl.ds(lg * 16, 16)
            tv = [t_v[f, lanes] for f in range(_EF)]

            def wm_body(q, c):
                acc = jnp.zeros((16,), jnp.float32)
                ma = jnp.zeros((16,), jnp.float32)
                for f in range(_EF):
                    v = em_v[q * _EF + f, lanes]
                    acc = acc + v * tv[f]
                    ma = jnp.maximum(ma, jnp.abs(v))
                wm = jnp.where(ma == 0.0, jnp.float32(-1e8), acc)
                wm_v[q, lanes] = (wm + jnp.float32(1e-8)
                                  + g_v[q, lanes]) / jnp.float32(0.1)
                return c

            lax.fori_loop(0, _QL, wm_body, 0)

            def max_body(q, m):
                return jnp.maximum(m, wm_v[q, lanes])

            m = lax.fori_loop(0, _QL, max_body,
                              jnp.full((16,), -jnp.inf, jnp.float32))

            def exp_body(q, s):
                e = jnp.exp(wm_v[q, lanes] - m)
                wm_v[q, lanes] = e
                return s + e

            s = lax.fori_loop(0, _QL, exp_body, jnp.zeros((16,), jnp.float32))
            inv = 1.0 / s

            def sel_body(q, accs):
                w = wm_v[q, lanes] * inv
                return tuple(accs[f] + w * em_v[q * _EF + f, lanes]
                             for f in range(_EF))

            accs = lax.fori_loop(
                0, _QL, sel_body,
                tuple(jnp.zeros((16,), jnp.float32) for _ in range(_EF)))
            for f in range(_EF):
                sel_v[f, lanes] = accs[f]
        pltpu.sync_copy(sel_v, out_hbm.at[:, pl.ds(c0, _CH)])
        return carry

    lax.fori_loop(0, cpw // _CH, chunk_body, 0)


@jax.jit
def kernel(vector_state, image_state, Wq, Wk, Wv, W1, b1, W2, b2):
    del Wv
    bsz = vector_state.shape[0]
    vst = vector_state.T
    imgt = image_state.transpose(1, 2, 3, 0).reshape(_GF, bsz)
    gt = _gumbel_noise(bsz).T
    wqt = Wq.T
    w1t = W1.T
    w2t = W2.T
    b1c = b1.reshape(_HID, 1)
    b2c = b2.reshape(_OUT, 1)

    btl = 512
    n_sc = bsz // _SC_FRAC
    n_tc = bsz - n_sc
    tc_blocks = n_tc // btl
    sc_blocks = n_sc // btl
    col = lambda i: (0, i)
    sc_col = lambda i: (0, tc_blocks + i)
    rep = lambda i: (0, 0)

    t_sc = pl.pallas_call(
        _t_body,
        grid=(sc_blocks,),
        in_specs=[
            pl.BlockSpec((_STATUS, btl), sc_col),
            pl.BlockSpec(wqt.shape, rep),
            pl.BlockSpec(Wk.shape, rep),
        ],
        out_specs=pl.BlockSpec((_EF, btl), col),
        out_shape=jax.ShapeDtypeStruct((_EF, n_sc), jnp.float32),
    )(vst, wqt, Wk)

    sc_routing = functools.partial(
        pl.kernel,
        mesh=plsc.VectorSubcoreMesh(core_axis_name="c", subcore_axis_name="s"),
        out_type=jax.ShapeDtypeStruct((_EF, n_sc), jnp.float32),
        scratch_types=[
            pltpu.VMEM((_EMD, _CH), jnp.float32),
            pltpu.VMEM((_EF, _CH), jnp.float32),
            pltpu.VMEM((_QL, _CH), jnp.float32),
            pltpu.VMEM((_QL, _CH), jnp.float32),
            pltpu.VMEM((_EF, _CH), jnp.float32),
        ],
    )(_sc_routing_body)
    selected_sc = sc_routing(vst, t_sc, gt)

    out_tc = pl.pallas_call(
        _fused_body,
        grid=(tc_blocks,),
        in_specs=[
            pl.BlockSpec((_STATUS + _EMD, btl), col),
            pl.BlockSpec((_GF, btl), col),
            pl.BlockSpec((_QL, btl), col),
            pl.BlockSpec(wqt.shape, rep),
            pl.BlockSpec(Wk.shape, rep),
            pl.BlockSpec(w1t.shape, rep),
            pl.BlockSpec(b1c.shape, rep),
            pl.BlockSpec(w2t.shape, rep),
            pl.BlockSpec(b2c.shape, rep),
        ],
        out_specs=pl.BlockSpec((_OUT, btl), col),
        out_shape=jax.ShapeDtypeStruct((_OUT, n_tc), jnp.float32),
    )(vst, imgt, gt, wqt, Wk, w1t, b1c, w2t, b2c)

    out_sc = pl.pallas_call(
        _mlp_body,
        grid=(sc_blocks,),
        in_specs=[
            pl.BlockSpec((_STATUS, btl), sc_col),
            pl.BlockSpec((_GF, btl), sc_col),
            pl.BlockSpec((_EF, btl), col),
            pl.BlockSpec(w1t.shape, rep),
            pl.BlockSpec(b1c.shape, rep),
            pl.BlockSpec(w2t.shape, rep),
            pl.BlockSpec(b2c.shape, rep),
        ],
        out_specs=pl.BlockSpec((_OUT, btl), col),
        out_shape=jax.ShapeDtypeStruct((_OUT, n_sc), jnp.float32),
    )(vst, imgt, selected_sc, w1t, b1c, w2t, b2c)

    return jnp.concatenate([out_tc, out_sc], axis=1).T


# speedup vs baseline: 1.7786x; 1.1053x over previous
import functools
import math

import jax
import jax.numpy as jnp
from jax import lax
from jax.experimental import pallas as pl
from jax.experimental.pallas import tpu as pltpu
from jax.experimental.pallas import tpu_sc as plsc

_STATUS = 128
_QL = 50
_EF = 16
_EMD = _QL * _EF
_H = 4
_DH = 32
_GF = 1600
_HID = 64
_OUT = 64

_DP = lax.Precision.DEFAULT

_NW = 32
_CH = 128
_SC_FRAC = 4


def _gumbel_noise(bsz):
    u = jax.random.uniform(jax.random.key(42), (bsz, _QL), dtype=jnp.float32)
    return -jnp.log(-jnp.log(u + 1e-20) + 1e-20)


def _routing_t(vst, wqt, wk):
    qf_t = jnp.dot(wqt, vst[:_STATUS], precision=_DP)
    return jnp.dot(wk, qf_t, precision=_DP) / jnp.float32(_H * math.sqrt(_DH))


def _select(em_t, t_t, g):
    col_f = lax.broadcasted_iota(jnp.int32, (_EMD, _EF), 0)
    row_f = lax.broadcasted_iota(jnp.int32, (_EMD, _EF), 1)
    rep = (lax.rem(col_f, _EF) == row_f).astype(jnp.float32)
    q_c = lax.broadcasted_iota(jnp.int32, (_QL, _EMD), 1)
    q_r = lax.broadcasted_iota(jnp.int32, (_QL, _EMD), 0)
    seg = (q_c // _EF == q_r).astype(jnp.float32)

    trep_t = jnp.dot(rep, t_t, precision=_DP)
    wm_t = jnp.dot(seg, em_t * trep_t, precision=_DP)
    nz = (em_t != 0.0).astype(jnp.float32)
    cnt = jnp.dot(seg, nz, precision=_DP)
    wm_t = jnp.where(cnt == 0.0, jnp.float32(-1e8), wm_t)

    logits = (wm_t + jnp.float32(1e-8) + g) / jnp.float32(0.1)
    m = jnp.max(logits, axis=0, keepdims=True)
    e = jnp.exp(logits - m)
    sel = e / jnp.sum(e, axis=0, keepdims=True)

    selrep_t = jnp.dot(seg.T, sel, precision=_DP)
    return jnp.dot(rep.T, em_t * selrep_t, precision=_DP)


def _fused_body(vs_ref, img_ref, g_ref, wqt_ref, wk_ref, w1t_ref, b1_ref,
                w2t_ref, b2_ref, out_ref):
    vst = vs_ref[...]
    t_t = _routing_t(vst, wqt_ref[...], wk_ref[...])
    selected_t = _select(vst[_STATUS:], t_t, g_ref[...])
    w1t = w1t_ref[...]
    h = (jnp.dot(w1t[:, :_STATUS], vst[:_STATUS], precision=_DP)
         + jnp.dot(w1t[:, _STATUS:_STATUS + _EF], selected_t, precision=_DP)
         + jnp.dot(w1t[:, _STATUS + _EF:], img_ref[...], precision=_DP)
         + b1_ref[...])
    h = jnp.maximum(h, 0.0)
    out_ref[...] = jnp.maximum(
        jnp.dot(w2t_ref[...], h, precision=_DP) + b2_ref[...], 0.0)


def _t_body(vs_ref, wqt_ref, wk_ref, t_ref):
    t_ref[...] = _routing_t(vs_ref[...], wqt_ref[...], wk_ref[...])


def _mlp_body(vs_ref, img_ref, sel_ref, w1t_ref, b1_ref, w2t_ref, b2_ref,
              out_ref):
    w1t = w1t_ref[...]
    h = jnp.maximum(
        jnp.dot(w1t[:, :_STATUS], vs_ref[...], precision=_DP)
        + jnp.dot(w1t[:, _STATUS:_STATUS + _EF], sel_ref[...], precision=_DP)
        + jnp.dot(w1t[:, _STATUS + _EF:], img_ref[...], precision=_DP)
        + b1_ref[...], 0.0)
    out_ref[...] = jnp.maximum(
        jnp.dot(w2t_ref[...], h, precision=_DP) + b2_ref[...], 0.0)


def _sc_routing_body(vst_hbm, tt_hbm, g_hbm, out_hbm, em_v, t_v, g_v, wm_v,
                     sel_v):
    nc = 2
    wid = lax.axis_index("s") * nc + lax.axis_index("c")
    n_sc = out_hbm.shape[1]
    sc0 = vst_hbm.shape[1] - n_sc
    cpw = n_sc // _NW
    base = wid * cpw

    def chunk_body(ci, carry):
        c0 = base + ci * _CH
        pltpu.sync_copy(
            vst_hbm.at[pl.ds(_STATUS, _EMD), pl.ds(sc0 + c0, _CH)], em_v)
        pltpu.sync_copy(tt_hbm.at[:, pl.ds(c0, _CH)], t_v)
        pltpu.sync_copy(g_hbm.at[:, pl.ds(sc0 + c0, _CH)], g_v)
        for lg in range(_CH // 16):
            lanes = pl.ds(lg * 16, 16)
            tv = [t_v[f, lanes] for f in range(_EF)]

            def wm_body(q, c):
                acc = jnp.zeros((16,), jnp.float32)
                ma = jnp.zeros((16,), jnp.float32)
                for f in range(_EF):
                    v = em_v[q * _EF + f, lanes]
                    acc = acc + v * tv[f]
                    ma = jnp.maximum(ma, jnp.abs(v))
                wm = jnp.where(ma == 0.0, jnp.float32(-1e8), acc)
                wm_v[q, lanes] = (wm + jnp.float32(1e-8)
                                  + g_v[q, lanes]) / jnp.float32(0.1)
                return c

            lax.fori_loop(0, _QL, wm_body, 0)

            def max_body(q, m):
                return jnp.maximum(m, wm_v[q, lanes])

            m = lax.fori_loop(0, _QL, max_body,
                              jnp.full((16,), -jnp.inf, jnp.float32))

            def exp_body(q, s):
                e = jnp.exp(wm_v[q, lanes] - m)
                wm_v[q, lanes] = e
                return s + e

            s = lax.fori_loop(0, _QL, exp_body, jnp.zeros((16,), jnp.float32))
            inv = 1.0 / s

            def sel_body(q, accs):
                w = wm_v[q, lanes] * inv
                return tuple(accs[f] + w * em_v[q * _EF + f, lanes]
                             for f in range(_EF))

            accs = lax.fori_loop(
                0, _QL, sel_body,
                tuple(jnp.zeros((16,), jnp.float32) for _ in range(_EF)))
            for f in range(_EF):
                sel_v[f, lanes] = accs[f]
        pltpu.sync_copy(sel_v, out_hbm.at[:, pl.ds(c0, _CH)])
        return carry

    lax.fori_loop(0, cpw // _CH, chunk_body, 0)


@jax.jit
def kernel(vector_state, image_state, Wq, Wk, Wv, W1, b1, W2, b2):
    del Wv
    bsz = vector_state.shape[0]
    vst = vector_state.T
    imgt = image_state.transpose(1, 2, 3, 0).reshape(_GF, bsz)
    gt = _gumbel_noise(bsz).T
    wqt = Wq.T
    w1t = W1.T
    w2t = W2.T
    b1c = b1.reshape(_HID, 1)
    b2c = b2.reshape(_OUT, 1)

    btl = 1024
    n_sc = bsz // _SC_FRAC
    n_tc = bsz - n_sc
    tc_blocks = n_tc // btl
    sc_blocks = n_sc // btl
    col = lambda i: (0, i)
    sc_col = lambda i: (0, tc_blocks + i)
    rep = lambda i: (0, 0)

    t_sc = pl.pallas_call(
        _t_body,
        grid=(sc_blocks,),
        in_specs=[
            pl.BlockSpec((_STATUS, btl), sc_col),
            pl.BlockSpec(wqt.shape, rep),
            pl.BlockSpec(Wk.shape, rep),
        ],
        out_specs=pl.BlockSpec((_EF, btl), col),
        out_shape=jax.ShapeDtypeStruct((_EF, n_sc), jnp.float32),
    )(vst, wqt, Wk)

    sc_routing = functools.partial(
        pl.kernel,
        mesh=plsc.VectorSubcoreMesh(core_axis_name="c", subcore_axis_name="s"),
        out_type=jax.ShapeDtypeStruct((_EF, n_sc), jnp.float32),
        scratch_types=[
            pltpu.VMEM((_EMD, _CH), jnp.float32),
            pltpu.VMEM((_EF, _CH), jnp.float32),
            pltpu.VMEM((_QL, _CH), jnp.float32),
            pltpu.VMEM((_QL, _CH), jnp.float32),
            pltpu.VMEM((_EF, _CH), jnp.float32),
        ],
    )(_sc_routing_body)
    selected_sc = sc_routing(vst, t_sc, gt)

    out_tc = pl.pallas_call(
        _fused_body,
        grid=(tc_blocks,),
        in_specs=[
            pl.BlockSpec((_STATUS + _EMD, btl), col),
            pl.BlockSpec((_GF, btl), col),
            pl.BlockSpec((_QL, btl), col),
            pl.BlockSpec(wqt.shape, rep),
            pl.BlockSpec(Wk.shape, rep),
            pl.BlockSpec(w1t.shape, rep),
            pl.BlockSpec(b1c.shape, rep),
            pl.BlockSpec(w2t.shape, rep),
            pl.BlockSpec(b2c.shape, rep),
        ],
        out_specs=pl.BlockSpec((_OUT, btl), col),
        out_shape=jax.ShapeDtypeStruct((_OUT, n_tc), jnp.float32),
    )(vst, imgt, gt, wqt, Wk, w1t, b1c, w2t, b2c)

    out_sc = pl.pallas_call(
        _mlp_body,
        grid=(sc_blocks,),
        in_specs=[
            pl.BlockSpec((_STATUS, btl), sc_col),
            pl.BlockSpec((_GF, btl), sc_col),
            pl.BlockSpec((_EF, btl), col),
            pl.BlockSpec(w1t.shape, rep),
            pl.BlockSpec(b1c.shape, rep),
            pl.BlockSpec(w2t.shape, rep),
            pl.BlockSpec(b2c.shape, rep),
        ],
        out_specs=pl.BlockSpec((_OUT, btl), col),
        out_shape=jax.ShapeDtypeStruct((_OUT, n_sc), jnp.float32),
    )(vst, imgt, selected_sc, w1t, b1c, w2t, b2c)

    return jnp.concatenate([out_tc, out_sc], axis=1).T


# speedup vs baseline: 1.8132x; 1.0194x over previous
import functools
import math

import jax
import jax.numpy as jnp
from jax import lax
from jax.experimental import pallas as pl
from jax.experimental.pallas import tpu as pltpu
from jax.experimental.pallas import tpu_sc as plsc

_STATUS = 128
_QL = 50
_EF = 16
_EMD = _QL * _EF
_H = 4
_DH = 32
_GF = 1600
_HID = 64
_OUT = 64

_DP = lax.Precision.DEFAULT

_NW = 32
_CH = 128
_SC_FRAC = 4


def _gumbel_noise(bsz):
    u = jax.random.uniform(jax.random.key(42), (bsz, _QL), dtype=jnp.float32)
    return -jnp.log(-jnp.log(u + 1e-20) + 1e-20)


def _routing_t(vst, wqt, wk):
    qf_t = jnp.dot(wqt, vst[:_STATUS], precision=_DP)
    return jnp.dot(wk, qf_t, precision=_DP) / jnp.float32(_H * math.sqrt(_DH))


def _select(em_t, t_t, g):
    col_f = lax.broadcasted_iota(jnp.int32, (_EMD, _EF), 0)
    row_f = lax.broadcasted_iota(jnp.int32, (_EMD, _EF), 1)
    rep = (lax.rem(col_f, _EF) == row_f).astype(jnp.float32)
    q_c = lax.broadcasted_iota(jnp.int32, (_QL, _EMD), 1)
    q_r = lax.broadcasted_iota(jnp.int32, (_QL, _EMD), 0)
    seg = (q_c // _EF == q_r).astype(jnp.float32)

    trep_t = jnp.dot(rep, t_t, precision=_DP)
    wm_t = jnp.dot(seg, em_t * trep_t, precision=_DP)
    nz = (em_t != 0.0).astype(jnp.float32)
    cnt = jnp.dot(seg, nz, precision=_DP)
    wm_t = jnp.where(cnt == 0.0, jnp.float32(-1e8), wm_t)

    logits = (wm_t + jnp.float32(1e-8) + g) / jnp.float32(0.1)
    m = jnp.max(logits, axis=0, keepdims=True)
    e = jnp.exp(logits - m)
    sel = e / jnp.sum(e, axis=0, keepdims=True)

    selrep_t = jnp.dot(seg.T, sel, precision=_DP)
    return jnp.dot(rep.T, em_t * selrep_t, precision=_DP)


def _fused_body(vs_ref, img_ref, g_ref, wqt_ref, wk_ref, w1t_ref, b1_ref,
                w2t_ref, b2_ref, out_ref):
    vst = vs_ref[...]
    t_t = _routing_t(vst, wqt_ref[...], wk_ref[...])
    selected_t = _select(vst[_STATUS:], t_t, g_ref[...])
    w1t = w1t_ref[...]
    h = (jnp.dot(w1t[:, :_STATUS], vst[:_STATUS], precision=_DP)
         + jnp.dot(w1t[:, _STATUS:_STATUS + _EF], selected_t, precision=_DP)
         + jnp.dot(w1t[:, _STATUS + _EF:], img_ref[...], precision=_DP)
         + b1_ref[...])
    h = jnp.maximum(h, 0.0)
    out_ref[...] = jnp.maximum(
        jnp.dot(w2t_ref[...], h, precision=_DP) + b2_ref[...], 0.0)


def _t_body(vs_ref, wqt_ref, wk_ref, t_ref):
    t_ref[...] = _routing_t(vs_ref[...], wqt_ref[...], wk_ref[...])


def _mlp_body(vs_ref, img_ref, sel_ref, w1t_ref, b1_ref, w2t_ref, b2_ref,
              out_ref):
    w1t = w1t_ref[...]
    h = jnp.maximum(
        jnp.dot(w1t[:, :_STATUS], vs_ref[...], precision=_DP)
        + jnp.dot(w1t[:, _STATUS:_STATUS + _EF], sel_ref[...], precision=_DP)
        + jnp.dot(w1t[:, _STATUS + _EF:], img_ref[...], precision=_DP)
        + b1_ref[...], 0.0)
    out_ref[...] = jnp.maximum(
        jnp.dot(w2t_ref[...], h, precision=_DP) + b2_ref[...], 0.0)


def _sc_routing_body(vst_hbm, tt_hbm, g_hbm, out_hbm, em_v, t_v, g_v, wm_v,
                     sel_v):
    nc = 2
    wid = lax.axis_index("s") * nc + lax.axis_index("c")
    n_sc = out_hbm.shape[1]
    sc0 = vst_hbm.shape[1] - n_sc
    cpw = n_sc // _NW
    base = wid * cpw

    def chunk_body(ci, carry):
        c0 = base + ci * _CH
        pltpu.sync_copy(
            vst_hbm.at[pl.ds(_STATUS, _EMD), pl.ds(sc0 + c0, _CH)], em_v)
        pltpu.sync_copy(tt_hbm.at[:, pl.ds(c0, _CH)], t_v)
        pltpu.sync_copy(g_hbm.at[:, pl.ds(sc0 + c0, _CH)], g_v)
        for lg in range(_CH // 16):
            lanes = pl.ds(lg * 16, 16)
            tv = [t_v[f, lanes] for f in range(_EF)]

            def wm_body(q, c):
                acc = jnp.zeros((16,), jnp.float32)
                ma = jnp.zeros((16,), jnp.float32)
                for f in range(_EF):
                    v = em_v[q * _EF + f, lanes]
                    acc = acc + v * tv[f]
                    ma = jnp.maximum(ma, jnp.abs(v))
                wm = jnp.where(ma == 0.0, jnp.float32(-1e8), acc)
                wm_v[q, lanes] = (wm + jnp.float32(1e-8)
                                  + g_v[q, lanes]) / jnp.float32(0.1)
                return c

            lax.fori_loop(0, _QL, wm_body, 0)

            def max_body(q, m):
                return jnp.maximum(m, wm_v[q, lanes])

            m = lax.fori_loop(0, _QL, max_body,
                              jnp.full((16,), -jnp.inf, jnp.float32))

            def exp_body(q, s):
                e = jnp.exp(wm_v[q, lanes] - m)
                wm_v[q, lanes] = e
                return s + e

            s = lax.fori_loop(0, _QL, exp_body, jnp.zeros((16,), jnp.float32))
            inv = 1.0 / s

            def sel_body(q, accs):
                w = wm_v[q, lanes] * inv
                return tuple(accs[f] + w * em_v[q * _EF + f, lanes]
                             for f in range(_EF))

            accs = lax.fori_loop(
                0, _QL, sel_body,
                tuple(jnp.zeros((16,), jnp.float32) for _ in range(_EF)))
            for f in range(_EF):
                sel_v[f, lanes] = accs[f]
        pltpu.sync_copy(sel_v, out_hbm.at[:, pl.ds(c0, _CH)])
        return carry

    lax.fori_loop(0, cpw // _CH, chunk_body, 0)


@jax.jit
def kernel(vector_state, image_state, Wq, Wk, Wv, W1, b1, W2, b2):
    del Wv
    bsz = vector_state.shape[0]
    vst = vector_state.T
    imgt = image_state.transpose(1, 2, 3, 0).reshape(_GF, bsz)
    gt = _gumbel_noise(bsz).T
    wqt = Wq.T
    w1t = W1.T
    w2t = W2.T
    b1c = b1.reshape(_HID, 1)
    b2c = b2.reshape(_OUT, 1)

    btl = 2048
    n_sc = bsz // _SC_FRAC
    n_tc = bsz - n_sc
    tc_blocks = n_tc // btl
    sc_blocks = n_sc // btl
    col = lambda i: (0, i)
    sc_col = lambda i: (0, tc_blocks + i)
    rep = lambda i: (0, 0)

    t_sc = pl.pallas_call(
        _t_body,
        grid=(sc_blocks,),
        in_specs=[
            pl.BlockSpec((_STATUS, btl), sc_col),
            pl.BlockSpec(wqt.shape, rep),
            pl.BlockSpec(Wk.shape, rep),
        ],
        out_specs=pl.BlockSpec((_EF, btl), col),
        out_shape=jax.ShapeDtypeStruct((_EF, n_sc), jnp.float32),
    )(vst, wqt, Wk)

    sc_routing = functools.partial(
        pl.kernel,
        mesh=plsc.VectorSubcoreMesh(core_axis_name="c", subcore_axis_name="s"),
        out_type=jax.ShapeDtypeStruct((_EF, n_sc), jnp.float32),
        scratch_types=[
            pltpu.VMEM((_EMD, _CH), jnp.float32),
            pltpu.VMEM((_EF, _CH), jnp.float32),
            pltpu.VMEM((_QL, _CH), jnp.float32),
            pltpu.VMEM((_QL, _CH), jnp.float32),
            pltpu.VMEM((_EF, _CH), jnp.float32),
        ],
    )(_sc_routing_body)
    selected_sc = sc_routing(vst, t_sc, gt)

    out_tc = pl.pallas_call(
        _fused_body,
        grid=(tc_blocks,),
        in_specs=[
            pl.BlockSpec((_STATUS + _EMD, btl), col),
            pl.BlockSpec((_GF, btl), col),
            pl.BlockSpec((_QL, btl), col),
            pl.BlockSpec(wqt.shape, rep),
            pl.BlockSpec(Wk.shape, rep),
            pl.BlockSpec(w1t.shape, rep),
            pl.BlockSpec(b1c.shape, rep),
            pl.BlockSpec(w2t.shape, rep),
            pl.BlockSpec(b2c.shape, rep),
        ],
        out_specs=pl.BlockSpec((_OUT, btl), col),
        out_shape=jax.ShapeDtypeStruct((_OUT, n_tc), jnp.float32),
    )(vst, imgt, gt, wqt, Wk, w1t, b1c, w2t, b2c)

    out_sc = pl.pallas_call(
        _mlp_body,
        grid=(sc_blocks,),
        in_specs=[
            pl.BlockSpec((_STATUS, btl), sc_col),
            pl.BlockSpec((_GF, btl), sc_col),
            pl.BlockSpec((_EF, btl), col),
            pl.BlockSpec(w1t.shape, rep),
            pl.BlockSpec(b1c.shape, rep),
            pl.BlockSpec(w2t.shape, rep),
            pl.BlockSpec(b2c.shape, rep),
        ],
        out_specs=pl.BlockSpec((_OUT, btl), col),
        out_shape=jax.ShapeDtypeStruct((_OUT, n_sc), jnp.float32),
    )(vst, imgt, selected_sc, w1t, b1c, w2t, b2c)

    return jnp.concatenate([out_tc, out_sc], axis=1).T
